# Initial kernel scaffold; baseline (speedup 1.0000x reference)
import jax
import jax.numpy as jnp
from jax.experimental import pallas as pl


def kernel(h, r, pos_t, neg_t, entity_embed, relation_embed):
    raise NotImplementedError("write your pallas kernel here")


# speedup vs baseline: 1.3582x; 1.3582x over previous
import functools

import jax
import jax.numpy as jnp
from jax import lax
from jax.experimental import pallas as pl
from jax.experimental.pallas import tpu as pltpu
from jax.experimental.pallas import tpu_sc as plsc

N_ENT = 1000000
N_REL = 1000
D = 64
ROW = 64 * D
B = 16384
LAM = 1e-4

NC, NS, L = 2, 16, 16
NW = NC * NS
IPW = B // NW
G = 8
NG = IPW // G


def _rsq_body(rel_ref, out_ref):
    blk = rel_ref[...]
    out_ref[...] = jnp.sum(blk * blk, axis=1).reshape(1, 1, 8)


def _make_rsq(relation_embed):
    rsq3 = pl.pallas_call(
        _rsq_body,
        grid=(N_REL // 8,),
        in_specs=[pl.BlockSpec((8, ROW), lambda i: (i, 0))],
        out_specs=pl.BlockSpec((1, 1, 8), lambda i: (i, 0, 0)),
        out_shape=jax.ShapeDtypeStruct((N_REL // 8, 1, 8), jnp.float32),
    )(relation_embed)
    return jnp.concatenate(
        [rsq3.reshape(N_REL), jnp.zeros((24,), jnp.float32)])


def _sc_body(h_hbm, r_hbm, p_hbm, n_hbm, ent_hbm, rel_hbm,
             x_hbm, part_hbm,
             hidx, ridx, pidx, nidx, rbuf, erows, xbuf, pbuf,
             sem0, sem1):
    wid = lax.axis_index("s") * NC + lax.axis_index("c")
    base = wid * IPW

    pltpu.sync_copy(h_hbm.at[pl.ds(base, IPW)], hidx)
    pltpu.sync_copy(r_hbm.at[pl.ds(base, IPW)], ridx)
    pltpu.sync_copy(p_hbm.at[pl.ds(base, IPW)], pidx)
    pltpu.sync_copy(n_hbm.at[pl.ds(base, IPW)], nidx)

    zero = jnp.zeros((L,), jnp.float32)
    for c in range(4):
        pbuf[c] = zero

    sems = (sem0, sem1)

    def fire(g, b):
        s = sems[b]
        pltpu.async_copy(rel_hbm.at[ridx.at[pl.ds(g * G, G)]], rbuf.at[b], s)
        pltpu.async_copy(ent_hbm.at[hidx.at[pl.ds(g * G, G)]],
                         erows.at[b, 0], s)
        pltpu.async_copy(ent_hbm.at[pidx.at[pl.ds(g * G, G)]],
                         erows.at[b, 1], s)
        pltpu.async_copy(ent_hbm.at[nidx.at[pl.ds(g * G, G)]],
                         erows.at[b, 2], s)

    def drain(g, b):
        s = sems[b]
        pltpu.make_async_copy(
            rel_hbm.at[ridx.at[pl.ds(g * G, G)]], rbuf.at[b], s).wait()
        pltpu.make_async_copy(
            ent_hbm.at[hidx.at[pl.ds(g * G, G)]], erows.at[b, 0], s).wait()
        pltpu.make_async_copy(
            ent_hbm.at[pidx.at[pl.ds(g * G, G)]], erows.at[b, 1], s).wait()
        pltpu.make_async_copy(
            ent_hbm.at[nidx.at[pl.ds(g * G, G)]], erows.at[b, 2], s).wait()

    def compute(g, b):
        for j in range(G):
            hrow = erows.at[b, 0, j]
            prow = erows.at[b, 1, j]
            nrow = erows.at[b, 2, j]
            hc = [hrow[pl.ds(16 * c, 16)] for c in range(4)]
            pc = [prow[pl.ds(16 * c, 16)] for c in range(4)]
            ncv = [nrow[pl.ds(16 * c, 16)] for c in range(4)]
            dc = [pc[c] - ncv[c] for c in range(4)]
            plsc.addupdate(pbuf.at[0],
                           hc[0] * hc[0] + hc[1] * hc[1]
                           + hc[2] * hc[2] + hc[3] * hc[3])
            plsc.addupdate(pbuf.at[1],
                           pc[0] * pc[0] + pc[1] * pc[1]
                           + pc[2] * pc[2] + pc[3] * pc[3])
            plsc.addupdate(pbuf.at[2],
                           ncv[0] * ncv[0] + ncv[1] * ncv[1]
                           + ncv[2] * ncv[2] + ncv[3] * ncv[3])

            row = rbuf.at[b, j]

            def iloop(c4, us):
                u0, u1, u2, u3 = us
                hv = hrow[pl.ds(c4 * 16, 16)]
                cbase = c4 * 1024
                for t in range(16):
                    hi = hv[t]
                    base_i = cbase + t * 64
                    u0 = u0 + hi * row[pl.ds(base_i, 16)]
                    u1 = u1 + hi * row[pl.ds(base_i + 16, 16)]
                    u2 = u2 + hi * row[pl.ds(base_i + 32, 16)]
                    u3 = u3 + hi * row[pl.ds(base_i + 48, 16)]
                return (u0, u1, u2, u3)

            u0, u1, u2, u3 = lax.fori_loop(
                0, 4, iloop, (zero, zero, zero, zero))
            xv = u0 * dc[0] + u1 * dc[1] + u2 * dc[2] + u3 * dc[3]
            xbuf[pl.ds((g * G + j) * L, L)] = xv

    fire(0, 0)

    def outer(gg, _):
        for bpar in range(2):
            g = gg * 2 + bpar

            @pl.when(g < NG)
            def _():
                drain(g, bpar)

                @pl.when(g + 1 < NG)
                def _():
                    fire(g + 1, 1 - bpar)

                compute(g, bpar)
        return _

    lax.fori_loop(0, (NG + 1) // 2, outer, None)

    pltpu.sync_copy(xbuf, x_hbm.at[pl.ds(base * L, IPW * L)])
    pltpu.sync_copy(pbuf, part_hbm.at[wid])


def _make_sc():
    mesh = plsc.VectorSubcoreMesh(
        core_axis_name="c", subcore_axis_name="s",
        num_cores=NC, num_subcores=NS)
    return pl.kernel(
        _sc_body,
        out_type=(jax.ShapeDtypeStruct((B * L,), jnp.float32),
                  jax.ShapeDtypeStruct((NW, 4, L), jnp.float32)),
        mesh=mesh,
        compiler_params=pltpu.CompilerParams(use_tc_tiling_on_sc=False),
        scratch_types=[
            pltpu.VMEM((IPW,), jnp.int32),
            pltpu.VMEM((IPW,), jnp.int32),
            pltpu.VMEM((IPW,), jnp.int32),
            pltpu.VMEM((IPW,), jnp.int32),
            pltpu.VMEM((2, G, ROW), jnp.float32),
            pltpu.VMEM((2, 3, G, D), jnp.float32),
            pltpu.VMEM((IPW * L,), jnp.float32),
            pltpu.VMEM((4, L), jnp.float32),
            pltpu.SemaphoreType.DMA,
            pltpu.SemaphoreType.DMA,
        ],
    )


def _fin_body(x_ref, p_ref, r_ref, rsq_ref, out_ref):
    x = jnp.sum(x_ref[...], axis=0, keepdims=True)
    sp = jnp.maximum(-x, 0.0) + jnp.log(1.0 + jnp.exp(-jnp.abs(x)))
    tl = jnp.sum(sp) * (1.0 / B)
    l2 = jnp.sum(p_ref[...]) * (LAM * 0.5 / B)
    iota_j = lax.broadcasted_iota(jnp.int32, (1, 1024), 1)
    rsq_row = rsq_ref[...]
    sr = jnp.float32(0.0)
    for k in range(B // 1024):
        rk = r_ref[:, k:k + 1]
        oh = (rk == iota_j).astype(jnp.float32)
        sr = sr + jnp.sum(oh * rsq_row)
    out_ref[0, 0] = tl + l2 + sr * (LAM * 0.5 / B)


def _finish(x, parts, r2d, rsq):
    out = pl.pallas_call(
        _fin_body,
        out_specs=pl.BlockSpec(memory_space=pltpu.SMEM),
        out_shape=jax.ShapeDtypeStruct((1, 1), jnp.float32),
    )(x.reshape(B, L).T, parts.reshape(NW * 4, L),
      r2d, rsq.reshape(1, 1024))
    return out[0, 0]


_SC_KERNEL = _make_sc()


@jax.jit
def kernel(h, r, pos_t, neg_t, entity_embed, relation_embed):
    h = h.astype(jnp.int32)
    r = r.astype(jnp.int32)
    pos_t = pos_t.astype(jnp.int32)
    neg_t = neg_t.astype(jnp.int32)
    rsq = _make_rsq(relation_embed)
    x, parts = _SC_KERNEL(h, r, pos_t, neg_t,
                          entity_embed, relation_embed)
    return _finish(x, parts, r.reshape(B // 1024, 1024).T, rsq)


# speedup vs baseline: 1.3871x; 1.0213x over previous
import functools

import jax
import jax.numpy as jnp
from jax import lax
from jax.experimental import pallas as pl
from jax.experimental.pallas import tpu as pltpu
from jax.experimental.pallas import tpu_sc as plsc

N_ENT = 1000000
N_REL = 1000
D = 64
ROW = 64 * D
B = 16384
LAM = 1e-4

NC, NS, L = 2, 16, 16
NW = NC * NS
IPW = B // NW
G = 8
NG = IPW // G


def _rsq_body(rel_ref, out_ref):
    blk = rel_ref[...]
    sums = jnp.sum(blk * blk, axis=1)
    lane0 = lax.broadcasted_iota(jnp.int32, (8, L), 1) == 0
    out_ref[...] = jnp.where(lane0, sums[:, None], 0.0).reshape(1, 8, L)


def _make_rsq(relation_embed):
    rsq3 = pl.pallas_call(
        _rsq_body,
        grid=(N_REL // 8,),
        in_specs=[pl.BlockSpec((8, ROW), lambda i: (i, 0))],
        out_specs=pl.BlockSpec((1, 8, L), lambda i: (i, 0, 0)),
        out_shape=jax.ShapeDtypeStruct((N_REL // 8, 8, L), jnp.float32),
    )(relation_embed)
    return rsq3.reshape(N_REL, L)


def _sc_body(h_hbm, r_hbm, p_hbm, n_hbm, ent_hbm, rel_hbm, rsq_hbm,
             x_hbm, part_hbm,
             hidx, ridx, pidx, nidx, rbuf, erows, rsqr, xbuf, pbuf,
             sem0, sem1):
    wid = lax.axis_index("s") * NC + lax.axis_index("c")
    base = wid * IPW

    pltpu.sync_copy(h_hbm.at[pl.ds(base, IPW)], hidx)
    pltpu.sync_copy(r_hbm.at[pl.ds(base, IPW)], ridx)
    pltpu.sync_copy(p_hbm.at[pl.ds(base, IPW)], pidx)
    pltpu.sync_copy(n_hbm.at[pl.ds(base, IPW)], nidx)

    zero = jnp.zeros((L,), jnp.float32)
    for c in range(4):
        pbuf[c] = zero

    sems = (sem0, sem1)

    def fire(g, b):
        s = sems[b]
        pltpu.async_copy(rel_hbm.at[ridx.at[pl.ds(g * G, G)]], rbuf.at[b], s)
        pltpu.async_copy(ent_hbm.at[hidx.at[pl.ds(g * G, G)]],
                         erows.at[b, 0], s)
        pltpu.async_copy(ent_hbm.at[pidx.at[pl.ds(g * G, G)]],
                         erows.at[b, 1], s)
        pltpu.async_copy(ent_hbm.at[nidx.at[pl.ds(g * G, G)]],
                         erows.at[b, 2], s)
        pltpu.async_copy(rsq_hbm.at[ridx.at[pl.ds(g * G, G)]],
                         rsqr.at[b], s)

    def drain(g, b):
        s = sems[b]
        pltpu.make_async_copy(
            rel_hbm.at[ridx.at[pl.ds(g * G, G)]], rbuf.at[b], s).wait()
        pltpu.make_async_copy(
            ent_hbm.at[hidx.at[pl.ds(g * G, G)]], erows.at[b, 0], s).wait()
        pltpu.make_async_copy(
            ent_hbm.at[pidx.at[pl.ds(g * G, G)]], erows.at[b, 1], s).wait()
        pltpu.make_async_copy(
            ent_hbm.at[nidx.at[pl.ds(g * G, G)]], erows.at[b, 2], s).wait()
        pltpu.make_async_copy(
            rsq_hbm.at[ridx.at[pl.ds(g * G, G)]], rsqr.at[b], s).wait()

    def compute(g, b):
        sr = rsqr[b, 0]
        for j in range(1, G):
            sr = sr + rsqr[b, j]
        plsc.addupdate(pbuf.at[3], sr)
        for j in range(G):
            hrow = erows.at[b, 0, j]
            prow = erows.at[b, 1, j]
            nrow = erows.at[b, 2, j]
            hc = [hrow[pl.ds(16 * c, 16)] for c in range(4)]
            pc = [prow[pl.ds(16 * c, 16)] for c in range(4)]
            ncv = [nrow[pl.ds(16 * c, 16)] for c in range(4)]
            dc = [pc[c] - ncv[c] for c in range(4)]
            plsc.addupdate(pbuf.at[0],
                           hc[0] * hc[0] + hc[1] * hc[1]
                           + hc[2] * hc[2] + hc[3] * hc[3])
            plsc.addupdate(pbuf.at[1],
                           pc[0] * pc[0] + pc[1] * pc[1]
                           + pc[2] * pc[2] + pc[3] * pc[3])
            plsc.addupdate(pbuf.at[2],
                           ncv[0] * ncv[0] + ncv[1] * ncv[1]
                           + ncv[2] * ncv[2] + ncv[3] * ncv[3])

            row = rbuf.at[b, j]

            def iloop(c4, us):
                u0, u1, u2, u3 = us
                hv = hrow[pl.ds(c4 * 16, 16)]
                cbase = c4 * 1024
                for t in range(16):
                    hi = hv[t]
                    base_i = cbase + t * 64
                    u0 = u0 + hi * row[pl.ds(base_i, 16)]
                    u1 = u1 + hi * row[pl.ds(base_i + 16, 16)]
                    u2 = u2 + hi * row[pl.ds(base_i + 32, 16)]
                    u3 = u3 + hi * row[pl.ds(base_i + 48, 16)]
                return (u0, u1, u2, u3)

            u0, u1, u2, u3 = lax.fori_loop(
                0, 4, iloop, (zero, zero, zero, zero))
            xv = u0 * dc[0] + u1 * dc[1] + u2 * dc[2] + u3 * dc[3]
            xbuf[pl.ds((g * G + j) * L, L)] = xv

    fire(0, 0)

    def outer(gg, _):
        for bpar in range(2):
            g = gg * 2 + bpar

            @pl.when(g < NG)
            def _():
                drain(g, bpar)

                @pl.when(g + 1 < NG)
                def _():
                    fire(g + 1, 1 - bpar)

                compute(g, bpar)
        return _

    lax.fori_loop(0, (NG + 1) // 2, outer, None)

    pltpu.sync_copy(xbuf, x_hbm.at[pl.ds(base * L, IPW * L)])
    pltpu.sync_copy(pbuf, part_hbm.at[wid])


def _make_sc():
    mesh = plsc.VectorSubcoreMesh(
        core_axis_name="c", subcore_axis_name="s",
        num_cores=NC, num_subcores=NS)
    return pl.kernel(
        _sc_body,
        out_type=(jax.ShapeDtypeStruct((B * L,), jnp.float32),
                  jax.ShapeDtypeStruct((NW, 4, L), jnp.float32)),
        mesh=mesh,
        compiler_params=pltpu.CompilerParams(use_tc_tiling_on_sc=False),
        scratch_types=[
            pltpu.VMEM((IPW,), jnp.int32),
            pltpu.VMEM((IPW,), jnp.int32),
            pltpu.VMEM((IPW,), jnp.int32),
            pltpu.VMEM((IPW,), jnp.int32),
            pltpu.VMEM((2, G, ROW), jnp.float32),
            pltpu.VMEM((2, 3, G, D), jnp.float32),
            pltpu.VMEM((2, G, L), jnp.float32),
            pltpu.VMEM((IPW * L,), jnp.float32),
            pltpu.VMEM((4, L), jnp.float32),
            pltpu.SemaphoreType.DMA,
            pltpu.SemaphoreType.DMA,
        ],
    )


def _fin_body(x_ref, p_ref, out_ref):
    xb = x_ref[...]
    mi = lax.broadcasted_iota(jnp.int32, (128, 128), 0)
    mj = lax.broadcasted_iota(jnp.int32, (128, 128), 1)
    m = jnp.where(mi // L == mj, 1.0, 0.0).astype(jnp.float32)
    y = jax.lax.dot(xb, m, precision=jax.lax.Precision.HIGHEST)
    x = y[:, 0:128 // L]
    sp = jnp.maximum(-x, 0.0) + jnp.log(1.0 + jnp.exp(-jnp.abs(x)))
    tl = jnp.sum(sp) * (1.0 / B)
    l2 = jnp.sum(p_ref[...]) * (LAM * 0.5 / B)
    out_ref[0, 0] = tl + l2


def _finish(x, parts):
    out = pl.pallas_call(
        _fin_body,
        out_specs=pl.BlockSpec(memory_space=pltpu.SMEM),
        out_shape=jax.ShapeDtypeStruct((1, 1), jnp.float32),
    )(x.reshape(B * L // 128, 128), parts.reshape(NW * 4, L))
    return out[0, 0]


_SC_KERNEL = _make_sc()


@jax.jit
def kernel(h, r, pos_t, neg_t, entity_embed, relation_embed):
    h = h.astype(jnp.int32)
    r = r.astype(jnp.int32)
    pos_t = pos_t.astype(jnp.int32)
    neg_t = neg_t.astype(jnp.int32)
    rsq = _make_rsq(relation_embed)
    x, parts = _SC_KERNEL(h, r, pos_t, neg_t,
                          entity_embed, relation_embed, rsq)
    return _finish(x, parts)
